# baseline (speedup 1.0000x reference)
import functools

import numpy as np

_IMGS_PER_STEP = 4

import jax
import jax.numpy as jnp
from jax import lax
from jax.experimental import pallas as pl
from jax.experimental.pallas import tpu as pltpu


def _edge_masks(H, W, C):
    col = np.arange(H * W) % W
    m = np.ones((2, H * W, 1), np.float32)
    m[0, col == 0, 0] = 0.0
    m[1, col == W - 1, 0] = 0.0
    return np.broadcast_to(m, (2, H * W, C)).copy()


def _conv_stats_kernel(x_ref, w_ref, m_ref, o_ref, s_ref, x3_ref, *, H, W):
    HW = H * W
    B = x_ref.shape[0]
    Cin = x_ref.shape[3]
    Cout = w_ref.shape[1] // 3
    zrow = jnp.zeros((W, Cin), jnp.float32)
    for b in range(B):
        x2 = x_ref[b].reshape(HW, Cin)
        x3_ref[:, 0:Cin] = jnp.concatenate(
            [zrow, x2[0:HW - W]], axis=0).astype(jnp.bfloat16)
        x3_ref[:, Cin:2 * Cin] = x2.astype(jnp.bfloat16)
        x3_ref[:, 2 * Cin:] = jnp.concatenate(
            [x2[W:HW], zrow], axis=0).astype(jnp.bfloat16)

        z = jnp.dot(x3_ref[...], w_ref[...],
                    preferred_element_type=jnp.float32)

        acc = z[:, Cout:2 * Cout]
        acc = acc + pltpu.roll(z[:, 0:Cout], 1, axis=0) * m_ref[0]
        acc = acc + pltpu.roll(z[:, 2 * Cout:], HW - 1, axis=0) * m_ref[1]

        o_ref[b] = acc.astype(jnp.bfloat16)
        s_ref[b, 0:1, :] = jnp.sum(acc, axis=0, keepdims=True)
        s_ref[b, 1:2, :] = jnp.sum(acc * acc, axis=0, keepdims=True)


def _norm_relu_kernel(s_ref, beta_ref, y_ref, o_ref, *, inv_count, eps):
    tot = jnp.sum(s_ref[...], axis=0)
    mean = tot[0:1, :] * inv_count
    var = jnp.maximum(tot[1:2, :] * inv_count - mean * mean, 0.0)
    scale = lax.rsqrt(var + eps)
    shift = beta_ref[...] - mean * scale
    y = y_ref[0].astype(jnp.float32)
    o_ref[...] = jnp.maximum(y * scale + shift, 0.0)[None]


@jax.jit
def _conv_block(x_nchw, weight_oihw, beta):
    eps = 1e-5
    N, Cin, H, W = x_nchw.shape
    Cout = weight_oihw.shape[0]
    HW = H * W

    x = jnp.transpose(x_nchw, (0, 2, 3, 1))
    w_cat = jnp.transpose(weight_oihw.astype(jnp.float32),
                          (2, 1, 3, 0)).reshape(3 * Cin, 3 * Cout)
    w_cat = w_cat.astype(jnp.bfloat16)
    masks = jnp.asarray(_edge_masks(H, W, Cout), dtype=jnp.float32)

    conv_flops = 2 * Cout * 9 * Cin * HW
    bs = _IMGS_PER_STEP if N % _IMGS_PER_STEP == 0 else 1

    y_raw, stats = pl.pallas_call(
        functools.partial(_conv_stats_kernel, H=H, W=W),
        out_shape=(
            jax.ShapeDtypeStruct((N, HW, Cout), jnp.bfloat16),
            jax.ShapeDtypeStruct((N, 2, Cout), jnp.float32),
        ),
        grid=(N // bs,),
        in_specs=[
            pl.BlockSpec((bs, H, W, Cin), lambda n: (n, 0, 0, 0)),
            pl.BlockSpec((3 * Cin, 3 * Cout), lambda n: (0, 0)),
            pl.BlockSpec((2, HW, Cout), lambda n: (0, 0, 0)),
        ],
        scratch_shapes=[pltpu.VMEM((HW, 3 * Cin), jnp.bfloat16)],
        out_specs=(
            pl.BlockSpec((bs, HW, Cout), lambda n: (n, 0, 0)),
            pl.BlockSpec((bs, 2, Cout), lambda n: (n, 0, 0)),
        ),
        compiler_params=pltpu.CompilerParams(
            dimension_semantics=("parallel",)),
        cost_estimate=pl.CostEstimate(
            flops=N * conv_flops,
            transcendentals=0,
            bytes_accessed=4 * N * Cin * HW + 2 * (3 * Cin * 3 * Cout
                                + N * Cout * HW) + 4 * (2 * HW * Cout
                                + N * Cout * 2)),
    )(x, w_cat, masks)

    y = pl.pallas_call(
        functools.partial(_norm_relu_kernel,
                          inv_count=1.0 / float(N * HW), eps=eps),
        out_shape=jax.ShapeDtypeStruct((N, HW, Cout), jnp.float32),
        grid=(N,),
        in_specs=[
            pl.BlockSpec((N, 2, Cout), lambda n: (0, 0, 0)),
            pl.BlockSpec((1, Cout), lambda n: (0, 0)),
            pl.BlockSpec((1, HW, Cout), lambda n: (n, 0, 0)),
        ],
        out_specs=pl.BlockSpec((1, HW, Cout), lambda n: (n, 0, 0)),
        compiler_params=pltpu.CompilerParams(
            dimension_semantics=("parallel",)),
        cost_estimate=pl.CostEstimate(
            flops=2 * N * Cout * HW,
            transcendentals=Cout,
            bytes_accessed=2 * N * Cout * HW + 4 * N * Cout * HW
                           + 4 * (N * Cout * 2 + Cout)),
    )(stats, beta.astype(jnp.float32).reshape(1, Cout), y_raw)

    return jnp.transpose(y.reshape(N, H, W, Cout), (0, 3, 1, 2))


def kernel(x_nchw, weight_oihw, beta):
    return _conv_block(x_nchw, weight_oihw, beta)


# speedup vs baseline: 1.0015x; 1.0015x over previous
import functools

import numpy as np

_IMGS_PER_STEP = 2

import jax
import jax.numpy as jnp
from jax import lax
from jax.experimental import pallas as pl
from jax.experimental.pallas import tpu as pltpu


def _edge_masks(H, W, C):
    col = np.arange(H * W) % W
    m = np.ones((2, H * W, 1), np.float32)
    m[0, col == 0, 0] = 0.0
    m[1, col == W - 1, 0] = 0.0
    return np.broadcast_to(m, (2, H * W, C)).copy()


def _conv_stats_kernel(x_ref, w_ref, m_ref, o_ref, s_ref, x3_ref, *, H, W):
    HW = H * W
    B = x_ref.shape[0]
    Cin = x_ref.shape[3]
    Cout = w_ref.shape[1] // 3
    zrow = jnp.zeros((W, Cin), jnp.float32)
    for b in range(B):
        x2 = x_ref[b].reshape(HW, Cin)
        x3_ref[:, 0:Cin] = jnp.concatenate(
            [zrow, x2[0:HW - W]], axis=0).astype(jnp.bfloat16)
        x3_ref[:, Cin:2 * Cin] = x2.astype(jnp.bfloat16)
        x3_ref[:, 2 * Cin:] = jnp.concatenate(
            [x2[W:HW], zrow], axis=0).astype(jnp.bfloat16)

        z = jnp.dot(x3_ref[...], w_ref[...],
                    preferred_element_type=jnp.float32)

        acc = z[:, Cout:2 * Cout]
        acc = acc + pltpu.roll(z[:, 0:Cout], 1, axis=0) * m_ref[0]
        acc = acc + pltpu.roll(z[:, 2 * Cout:], HW - 1, axis=0) * m_ref[1]

        o_ref[b] = acc.astype(jnp.bfloat16)
        s_ref[b, 0:1, :] = jnp.sum(acc, axis=0, keepdims=True)
        s_ref[b, 1:2, :] = jnp.sum(acc * acc, axis=0, keepdims=True)


def _norm_relu_kernel(s_ref, beta_ref, y_ref, o_ref, *, inv_count, eps):
    tot = jnp.sum(s_ref[...], axis=0)
    mean = tot[0:1, :] * inv_count
    var = jnp.maximum(tot[1:2, :] * inv_count - mean * mean, 0.0)
    scale = lax.rsqrt(var + eps)
    shift = beta_ref[...] - mean * scale
    y = y_ref[0].astype(jnp.float32)
    o_ref[...] = jnp.maximum(y * scale + shift, 0.0)[None]


@jax.jit
def _conv_block(x_nchw, weight_oihw, beta):
    eps = 1e-5
    N, Cin, H, W = x_nchw.shape
    Cout = weight_oihw.shape[0]
    HW = H * W

    x = jnp.transpose(x_nchw, (0, 2, 3, 1))
    w_cat = jnp.transpose(weight_oihw.astype(jnp.float32),
                          (2, 1, 3, 0)).reshape(3 * Cin, 3 * Cout)
    w_cat = w_cat.astype(jnp.bfloat16)
    masks = jnp.asarray(_edge_masks(H, W, Cout), dtype=jnp.float32)

    conv_flops = 2 * Cout * 9 * Cin * HW
    bs = _IMGS_PER_STEP if N % _IMGS_PER_STEP == 0 else 1

    y_raw, stats = pl.pallas_call(
        functools.partial(_conv_stats_kernel, H=H, W=W),
        out_shape=(
            jax.ShapeDtypeStruct((N, HW, Cout), jnp.bfloat16),
            jax.ShapeDtypeStruct((N, 2, Cout), jnp.float32),
        ),
        grid=(N // bs,),
        in_specs=[
            pl.BlockSpec((bs, H, W, Cin), lambda n: (n, 0, 0, 0)),
            pl.BlockSpec((3 * Cin, 3 * Cout), lambda n: (0, 0)),
            pl.BlockSpec((2, HW, Cout), lambda n: (0, 0, 0)),
        ],
        scratch_shapes=[pltpu.VMEM((HW, 3 * Cin), jnp.bfloat16)],
        out_specs=(
            pl.BlockSpec((bs, HW, Cout), lambda n: (n, 0, 0)),
            pl.BlockSpec((bs, 2, Cout), lambda n: (n, 0, 0)),
        ),
        compiler_params=pltpu.CompilerParams(
            dimension_semantics=("arbitrary",)),
        cost_estimate=pl.CostEstimate(
            flops=N * conv_flops,
            transcendentals=0,
            bytes_accessed=4 * N * Cin * HW + 2 * (3 * Cin * 3 * Cout
                                + N * Cout * HW) + 4 * (2 * HW * Cout
                                + N * Cout * 2)),
    )(x, w_cat, masks)

    y = pl.pallas_call(
        functools.partial(_norm_relu_kernel,
                          inv_count=1.0 / float(N * HW), eps=eps),
        out_shape=jax.ShapeDtypeStruct((N, HW, Cout), jnp.float32),
        grid=(N,),
        in_specs=[
            pl.BlockSpec((N, 2, Cout), lambda n: (0, 0, 0)),
            pl.BlockSpec((1, Cout), lambda n: (0, 0)),
            pl.BlockSpec((1, HW, Cout), lambda n: (n, 0, 0)),
        ],
        out_specs=pl.BlockSpec((1, HW, Cout), lambda n: (n, 0, 0)),
        compiler_params=pltpu.CompilerParams(
            dimension_semantics=("arbitrary",)),
        cost_estimate=pl.CostEstimate(
            flops=2 * N * Cout * HW,
            transcendentals=Cout,
            bytes_accessed=2 * N * Cout * HW + 4 * N * Cout * HW
                           + 4 * (N * Cout * 2 + Cout)),
    )(stats, beta.astype(jnp.float32).reshape(1, Cout), y_raw)

    return jnp.transpose(y.reshape(N, H, W, Cout), (0, 3, 1, 2))


def kernel(x_nchw, weight_oihw, beta):
    return _conv_block(x_nchw, weight_oihw, beta)


# speedup vs baseline: 1.0162x; 1.0147x over previous
import functools

import numpy as np

_IMGS_PER_STEP = 2

import jax
import jax.numpy as jnp
from jax import lax
from jax.experimental import pallas as pl
from jax.experimental.pallas import tpu as pltpu


def _edge_masks(H, W, C):
    col = np.arange(H * W) % W
    m = np.ones((2, H * W, 1), np.float32)
    m[0, col == 0, 0] = 0.0
    m[1, col == W - 1, 0] = 0.0
    return np.broadcast_to(m, (2, H * W, C)).copy()


def _conv_stats_kernel(x_ref, w_ref, m_ref, o_ref, s_ref, x3_ref, *, H, W):
    HW = H * W
    B = x_ref.shape[0]
    Cin = x_ref.shape[3]
    Cout = w_ref.shape[1] // 3
    zrow = jnp.zeros((W, Cin), jnp.float32)
    for b in range(B):
        x2 = x_ref[b].reshape(HW, Cin)
        x3_ref[b, :, 0:Cin] = jnp.concatenate(
            [zrow, x2[0:HW - W]], axis=0).astype(jnp.bfloat16)
        x3_ref[b, :, Cin:2 * Cin] = x2.astype(jnp.bfloat16)
        x3_ref[b, :, 2 * Cin:] = jnp.concatenate(
            [x2[W:HW], zrow], axis=0).astype(jnp.bfloat16)

        z = jnp.dot(x3_ref[b], w_ref[...],
                    preferred_element_type=jnp.float32)

        acc = z[:, Cout:2 * Cout]
        acc = acc + pltpu.roll(z[:, 0:Cout], 1, axis=0) * m_ref[0]
        acc = acc + pltpu.roll(z[:, 2 * Cout:], HW - 1, axis=0) * m_ref[1]

        o_ref[b] = acc.astype(jnp.bfloat16)
        s_ref[b, 0:1, :] = jnp.sum(acc, axis=0, keepdims=True)
        s_ref[b, 1:2, :] = jnp.sum(acc * acc, axis=0, keepdims=True)


def _norm_relu_kernel(s_ref, beta_ref, y_ref, o_ref, *, inv_count, eps):
    tot = jnp.sum(s_ref[...], axis=0)
    mean = tot[0:1, :] * inv_count
    var = jnp.maximum(tot[1:2, :] * inv_count - mean * mean, 0.0)
    scale = lax.rsqrt(var + eps)
    shift = beta_ref[...] - mean * scale
    y = y_ref[0].astype(jnp.float32)
    o_ref[...] = jnp.maximum(y * scale + shift, 0.0)[None]


@jax.jit
def _conv_block(x_nchw, weight_oihw, beta):
    eps = 1e-5
    N, Cin, H, W = x_nchw.shape
    Cout = weight_oihw.shape[0]
    HW = H * W

    x = jnp.transpose(x_nchw, (0, 2, 3, 1))
    w_cat = jnp.transpose(weight_oihw.astype(jnp.float32),
                          (2, 1, 3, 0)).reshape(3 * Cin, 3 * Cout)
    w_cat = w_cat.astype(jnp.bfloat16)
    masks = jnp.asarray(_edge_masks(H, W, Cout), dtype=jnp.float32)

    conv_flops = 2 * Cout * 9 * Cin * HW
    bs = _IMGS_PER_STEP if N % _IMGS_PER_STEP == 0 else 1

    y_raw, stats = pl.pallas_call(
        functools.partial(_conv_stats_kernel, H=H, W=W),
        out_shape=(
            jax.ShapeDtypeStruct((N, HW, Cout), jnp.bfloat16),
            jax.ShapeDtypeStruct((N, 2, Cout), jnp.float32),
        ),
        grid=(N // bs,),
        in_specs=[
            pl.BlockSpec((bs, H, W, Cin), lambda n: (n, 0, 0, 0)),
            pl.BlockSpec((3 * Cin, 3 * Cout), lambda n: (0, 0)),
            pl.BlockSpec((2, HW, Cout), lambda n: (0, 0, 0)),
        ],
        scratch_shapes=[pltpu.VMEM((bs, HW, 3 * Cin), jnp.bfloat16)],
        out_specs=(
            pl.BlockSpec((bs, HW, Cout), lambda n: (n, 0, 0)),
            pl.BlockSpec((bs, 2, Cout), lambda n: (n, 0, 0)),
        ),
        compiler_params=pltpu.CompilerParams(
            dimension_semantics=("parallel",)),
        cost_estimate=pl.CostEstimate(
            flops=N * conv_flops,
            transcendentals=0,
            bytes_accessed=4 * N * Cin * HW + 2 * (3 * Cin * 3 * Cout
                                + N * Cout * HW) + 4 * (2 * HW * Cout
                                + N * Cout * 2)),
    )(x, w_cat, masks)

    y = pl.pallas_call(
        functools.partial(_norm_relu_kernel,
                          inv_count=1.0 / float(N * HW), eps=eps),
        out_shape=jax.ShapeDtypeStruct((N, HW, Cout), jnp.float32),
        grid=(N,),
        in_specs=[
            pl.BlockSpec((N, 2, Cout), lambda n: (0, 0, 0)),
            pl.BlockSpec((1, Cout), lambda n: (0, 0)),
            pl.BlockSpec((1, HW, Cout), lambda n: (n, 0, 0)),
        ],
        out_specs=pl.BlockSpec((1, HW, Cout), lambda n: (n, 0, 0)),
        compiler_params=pltpu.CompilerParams(
            dimension_semantics=("parallel",)),
        cost_estimate=pl.CostEstimate(
            flops=2 * N * Cout * HW,
            transcendentals=Cout,
            bytes_accessed=2 * N * Cout * HW + 4 * N * Cout * HW
                           + 4 * (N * Cout * 2 + Cout)),
    )(stats, beta.astype(jnp.float32).reshape(1, Cout), y_raw)

    return jnp.transpose(y.reshape(N, H, W, Cout), (0, 3, 1, 2))


def kernel(x_nchw, weight_oihw, beta):
    return _conv_block(x_nchw, weight_oihw, beta)


# speedup vs baseline: 1.1108x; 1.0931x over previous
import functools

import numpy as np

_IMGS_PER_STEP = 2

import jax
import jax.numpy as jnp
from jax import lax
from jax.experimental import pallas as pl
from jax.experimental.pallas import tpu as pltpu


def _edge_masks(H, W, C):
    col = np.arange(H * W) % W
    m = np.ones((2, H * W, 1), np.float32)
    m[0, col == 0, 0] = 0.0
    m[1, col == W - 1, 0] = 0.0
    return np.broadcast_to(m, (2, H * W, C)).copy()


def _conv_stats_kernel(x_ref, w_ref, m_ref, o_ref, s_ref, x3_ref, *, H, W):
    HW = H * W
    B = x_ref.shape[0]
    Cin = x_ref.shape[3]
    Cout = w_ref.shape[1] // 3
    zrow = jnp.zeros((W, Cin), jnp.float32)
    for b in range(B):
        x2 = x_ref[b].reshape(HW, Cin)
        x3_ref[b, :, 0:Cin] = jnp.concatenate(
            [zrow, x2[0:HW - W]], axis=0).astype(jnp.bfloat16)
        x3_ref[b, :, Cin:2 * Cin] = x2.astype(jnp.bfloat16)
        x3_ref[b, :, 2 * Cin:] = jnp.concatenate(
            [x2[W:HW], zrow], axis=0).astype(jnp.bfloat16)

        z = jnp.dot(x3_ref[b], w_ref[...],
                    preferred_element_type=jnp.float32)

        acc = z[:, Cout:2 * Cout]
        acc = acc + pltpu.roll(z[:, 0:Cout], 1, axis=0) * m_ref[0]
        acc = acc + pltpu.roll(z[:, 2 * Cout:], HW - 1, axis=0) * m_ref[1]

        o_ref[b] = acc.astype(jnp.bfloat16)
        s_ref[b, 0:1, :] = jnp.sum(acc, axis=0, keepdims=True)
        s_ref[b, 1:2, :] = jnp.sum(acc * acc, axis=0, keepdims=True)


def _norm_relu_kernel(s_ref, beta_ref, y_ref, o_ref, *, inv_count, eps):
    tot = jnp.sum(s_ref[...], axis=0)
    mean = tot[0:1, :] * inv_count
    var = jnp.maximum(tot[1:2, :] * inv_count - mean * mean, 0.0)
    scale = lax.rsqrt(var + eps)
    shift = beta_ref[...] - mean * scale
    for b in range(y_ref.shape[0]):
        y = y_ref[b].astype(jnp.float32)
        o_ref[b] = jnp.maximum(y * scale + shift, 0.0)


@jax.jit
def _conv_block(x_nchw, weight_oihw, beta):
    eps = 1e-5
    N, Cin, H, W = x_nchw.shape
    Cout = weight_oihw.shape[0]
    HW = H * W

    x = jnp.transpose(x_nchw, (0, 2, 3, 1))
    w_cat = jnp.transpose(weight_oihw.astype(jnp.float32),
                          (2, 1, 3, 0)).reshape(3 * Cin, 3 * Cout)
    w_cat = w_cat.astype(jnp.bfloat16)
    masks = jnp.asarray(_edge_masks(H, W, Cout), dtype=jnp.float32)

    conv_flops = 2 * Cout * 9 * Cin * HW
    bs = _IMGS_PER_STEP if N % _IMGS_PER_STEP == 0 else 1

    y_raw, stats = pl.pallas_call(
        functools.partial(_conv_stats_kernel, H=H, W=W),
        out_shape=(
            jax.ShapeDtypeStruct((N, HW, Cout), jnp.bfloat16),
            jax.ShapeDtypeStruct((N, 2, Cout), jnp.float32),
        ),
        grid=(N // bs,),
        in_specs=[
            pl.BlockSpec((bs, H, W, Cin), lambda n: (n, 0, 0, 0)),
            pl.BlockSpec((3 * Cin, 3 * Cout), lambda n: (0, 0)),
            pl.BlockSpec((2, HW, Cout), lambda n: (0, 0, 0)),
        ],
        scratch_shapes=[pltpu.VMEM((bs, HW, 3 * Cin), jnp.bfloat16)],
        out_specs=(
            pl.BlockSpec((bs, HW, Cout), lambda n: (n, 0, 0)),
            pl.BlockSpec((bs, 2, Cout), lambda n: (n, 0, 0)),
        ),
        compiler_params=pltpu.CompilerParams(
            dimension_semantics=("parallel",)),
        cost_estimate=pl.CostEstimate(
            flops=N * conv_flops,
            transcendentals=0,
            bytes_accessed=4 * N * Cin * HW + 2 * (3 * Cin * 3 * Cout
                                + N * Cout * HW) + 4 * (2 * HW * Cout
                                + N * Cout * 2)),
    )(x, w_cat, masks)

    y = pl.pallas_call(
        functools.partial(_norm_relu_kernel,
                          inv_count=1.0 / float(N * HW), eps=eps),
        out_shape=jax.ShapeDtypeStruct((N, HW, Cout), jnp.float32),
        grid=(N // bs,),
        in_specs=[
            pl.BlockSpec((N, 2, Cout), lambda n: (0, 0, 0)),
            pl.BlockSpec((1, Cout), lambda n: (0, 0)),
            pl.BlockSpec((bs, HW, Cout), lambda n: (n, 0, 0)),
        ],
        out_specs=pl.BlockSpec((bs, HW, Cout), lambda n: (n, 0, 0)),
        compiler_params=pltpu.CompilerParams(
            dimension_semantics=("parallel",)),
        cost_estimate=pl.CostEstimate(
            flops=2 * N * Cout * HW,
            transcendentals=Cout,
            bytes_accessed=2 * N * Cout * HW + 4 * N * Cout * HW
                           + 4 * (N * Cout * 2 + Cout)),
    )(stats, beta.astype(jnp.float32).reshape(1, Cout), y_raw)

    return jnp.transpose(y.reshape(N, H, W, Cout), (0, 3, 1, 2))


def kernel(x_nchw, weight_oihw, beta):
    return _conv_block(x_nchw, weight_oihw, beta)


# speedup vs baseline: 1.1344x; 1.0213x over previous
import functools

import numpy as np

_IMGS_PER_STEP = 2

import jax
import jax.numpy as jnp
from jax import lax
from jax.experimental import pallas as pl
from jax.experimental.pallas import tpu as pltpu


def _edge_masks(H, W, C):
    col = np.arange(H * W) % W
    m = np.ones((2, H * W, 1), np.float32)
    m[0, col == 0, 0] = 0.0
    m[1, col == W - 1, 0] = 0.0
    return np.broadcast_to(m, (2, H * W, C)).copy()


def _conv_stats_kernel(x_ref, w_ref, m_ref, o_ref, s_ref, x3_ref, *, H, W):
    HW = H * W
    B = x_ref.shape[0]
    Cin = x_ref.shape[3]
    Cout = w_ref.shape[1] // 3
    zrow = jnp.zeros((W, Cin), jnp.float32)
    for b in range(B):
        x2 = x_ref[b].reshape(HW, Cin)
        x3_ref[b, :, 0:Cin] = jnp.concatenate(
            [zrow, x2[0:HW - W]], axis=0).astype(jnp.bfloat16)
        x3_ref[b, :, Cin:2 * Cin] = x2.astype(jnp.bfloat16)
        x3_ref[b, :, 2 * Cin:] = jnp.concatenate(
            [x2[W:HW], zrow], axis=0).astype(jnp.bfloat16)

        z = jnp.dot(x3_ref[b], w_ref[...],
                    preferred_element_type=jnp.float32)

        acc = z[:, Cout:2 * Cout]
        acc = acc + pltpu.roll(z[:, 0:Cout], 1, axis=0) * m_ref[0]
        acc = acc + pltpu.roll(z[:, 2 * Cout:], HW - 1, axis=0) * m_ref[1]

        o_ref[b] = acc.astype(jnp.bfloat16)
        s_ref[b, 0:1, :] = jnp.sum(acc, axis=0, keepdims=True)
        s_ref[b, 1:2, :] = jnp.sum(acc * acc, axis=0, keepdims=True)


def _norm_relu_kernel(s_ref, beta_ref, y_ref, o_ref, *, inv_count, eps):
    tot = jnp.sum(s_ref[...], axis=0)
    mean = tot[0:1, :] * inv_count
    var = jnp.maximum(tot[1:2, :] * inv_count - mean * mean, 0.0)
    scale = lax.rsqrt(var + eps)
    shift = beta_ref[...] - mean * scale
    for b in range(y_ref.shape[0]):
        y = y_ref[b].astype(jnp.float32)
        o_ref[b] = jnp.maximum(y * scale + shift, 0.0)


@jax.jit
def _conv_block(x_nchw, weight_oihw, beta):
    eps = 1e-5
    N, Cin, H, W = x_nchw.shape
    Cout = weight_oihw.shape[0]
    HW = H * W

    x = jnp.transpose(x_nchw, (0, 2, 3, 1))
    w_cat = jnp.transpose(weight_oihw.astype(jnp.float32),
                          (2, 1, 3, 0)).reshape(3 * Cin, 3 * Cout)
    w_cat = w_cat.astype(jnp.bfloat16)
    masks = jnp.asarray(_edge_masks(H, W, Cout), dtype=jnp.float32)

    conv_flops = 2 * Cout * 9 * Cin * HW
    bs = _IMGS_PER_STEP if N % _IMGS_PER_STEP == 0 else 1
    bs2 = 4 if N % 4 == 0 else 1

    y_raw, stats = pl.pallas_call(
        functools.partial(_conv_stats_kernel, H=H, W=W),
        out_shape=(
            jax.ShapeDtypeStruct((N, HW, Cout), jnp.bfloat16),
            jax.ShapeDtypeStruct((N, 2, Cout), jnp.float32),
        ),
        grid=(N // bs,),
        in_specs=[
            pl.BlockSpec((bs, H, W, Cin), lambda n: (n, 0, 0, 0)),
            pl.BlockSpec((3 * Cin, 3 * Cout), lambda n: (0, 0)),
            pl.BlockSpec((2, HW, Cout), lambda n: (0, 0, 0)),
        ],
        scratch_shapes=[pltpu.VMEM((bs, HW, 3 * Cin), jnp.bfloat16)],
        out_specs=(
            pl.BlockSpec((bs, HW, Cout), lambda n: (n, 0, 0)),
            pl.BlockSpec((bs, 2, Cout), lambda n: (n, 0, 0)),
        ),
        compiler_params=pltpu.CompilerParams(
            dimension_semantics=("parallel",)),
        cost_estimate=pl.CostEstimate(
            flops=N * conv_flops,
            transcendentals=0,
            bytes_accessed=4 * N * Cin * HW + 2 * (3 * Cin * 3 * Cout
                                + N * Cout * HW) + 4 * (2 * HW * Cout
                                + N * Cout * 2)),
    )(x, w_cat, masks)

    y = pl.pallas_call(
        functools.partial(_norm_relu_kernel,
                          inv_count=1.0 / float(N * HW), eps=eps),
        out_shape=jax.ShapeDtypeStruct((N, HW, Cout), jnp.float32),
        grid=(N // bs2,),
        in_specs=[
            pl.BlockSpec((N, 2, Cout), lambda n: (0, 0, 0)),
            pl.BlockSpec((1, Cout), lambda n: (0, 0)),
            pl.BlockSpec((bs2, HW, Cout), lambda n: (n, 0, 0)),
        ],
        out_specs=pl.BlockSpec((bs2, HW, Cout), lambda n: (n, 0, 0)),
        compiler_params=pltpu.CompilerParams(
            dimension_semantics=("parallel",)),
        cost_estimate=pl.CostEstimate(
            flops=2 * N * Cout * HW,
            transcendentals=Cout,
            bytes_accessed=2 * N * Cout * HW + 4 * N * Cout * HW
                           + 4 * (N * Cout * 2 + Cout)),
    )(stats, beta.astype(jnp.float32).reshape(1, Cout), y_raw)

    return jnp.transpose(y.reshape(N, H, W, Cout), (0, 3, 1, 2))


def kernel(x_nchw, weight_oihw, beta):
    return _conv_block(x_nchw, weight_oihw, beta)


# speedup vs baseline: 1.1607x; 1.0231x over previous
import functools

import numpy as np

_IMGS_PER_STEP = 2

import jax
import jax.numpy as jnp
from jax import lax
from jax.experimental import pallas as pl
from jax.experimental.pallas import tpu as pltpu


def _edge_masks(H, W, C):
    col = np.arange(H * W) % W
    m = np.ones((2, H * W, 1), np.float32)
    m[0, col == 0, 0] = 0.0
    m[1, col == W - 1, 0] = 0.0
    return np.broadcast_to(m, (2, H * W, C)).copy()


def _conv_stats_kernel(x_ref, w_ref, m_ref, o_ref, s_ref, x3_ref, *, H, W):
    HW = H * W
    B = x_ref.shape[0]
    Cin = x_ref.shape[3]
    Cout = w_ref.shape[1] // 3
    zrow = jnp.zeros((W, Cin), jnp.float32)
    for b in range(B):
        x2 = x_ref[b].reshape(HW, Cin)
        x3_ref[b, :, 0:Cin] = jnp.concatenate(
            [zrow, x2[0:HW - W]], axis=0).astype(jnp.bfloat16)
        x3_ref[b, :, Cin:2 * Cin] = x2.astype(jnp.bfloat16)
        x3_ref[b, :, 2 * Cin:] = jnp.concatenate(
            [x2[W:HW], zrow], axis=0).astype(jnp.bfloat16)

        z = jnp.dot(x3_ref[b], w_ref[...],
                    preferred_element_type=jnp.float32)

        acc = z[:, Cout:2 * Cout]
        acc = acc + pltpu.roll(z[:, 0:Cout], 1, axis=0) * m_ref[0]
        acc = acc + pltpu.roll(z[:, 2 * Cout:], HW - 1, axis=0) * m_ref[1]

        o_ref[b] = acc.astype(jnp.bfloat16)
        s_ref[b, 0:1, :] = jnp.sum(acc, axis=0, keepdims=True)
        s_ref[b, 1:2, :] = jnp.sum(acc * acc, axis=0, keepdims=True)


def _norm_relu_kernel(s_ref, beta_ref, y_ref, o_ref, *, inv_count, eps):
    tot = jnp.sum(s_ref[...], axis=0)
    mean = tot[0:1, :] * inv_count
    var = jnp.maximum(tot[1:2, :] * inv_count - mean * mean, 0.0)
    scale = lax.rsqrt(var + eps)
    shift = beta_ref[...] - mean * scale
    for b in range(y_ref.shape[0]):
        y = y_ref[b].astype(jnp.float32)
        o_ref[b] = jnp.maximum(y * scale + shift, 0.0)


@jax.jit
def _conv_block(x_nchw, weight_oihw, beta):
    eps = 1e-5
    N, Cin, H, W = x_nchw.shape
    Cout = weight_oihw.shape[0]
    HW = H * W

    x = jnp.transpose(x_nchw, (0, 2, 3, 1))
    w_cat = jnp.transpose(weight_oihw.astype(jnp.float32),
                          (2, 1, 3, 0)).reshape(3 * Cin, 3 * Cout)
    w_cat = w_cat.astype(jnp.bfloat16)
    masks = jnp.asarray(_edge_masks(H, W, Cout), dtype=jnp.float32)

    conv_flops = 2 * Cout * 9 * Cin * HW
    bs = _IMGS_PER_STEP if N % _IMGS_PER_STEP == 0 else 1
    bs2 = 8 if N % 8 == 0 else 1

    y_raw, stats = pl.pallas_call(
        functools.partial(_conv_stats_kernel, H=H, W=W),
        out_shape=(
            jax.ShapeDtypeStruct((N, HW, Cout), jnp.bfloat16),
            jax.ShapeDtypeStruct((N, 2, Cout), jnp.float32),
        ),
        grid=(N // bs,),
        in_specs=[
            pl.BlockSpec((bs, H, W, Cin), lambda n: (n, 0, 0, 0)),
            pl.BlockSpec((3 * Cin, 3 * Cout), lambda n: (0, 0)),
            pl.BlockSpec((2, HW, Cout), lambda n: (0, 0, 0)),
        ],
        scratch_shapes=[pltpu.VMEM((bs, HW, 3 * Cin), jnp.bfloat16)],
        out_specs=(
            pl.BlockSpec((bs, HW, Cout), lambda n: (n, 0, 0)),
            pl.BlockSpec((bs, 2, Cout), lambda n: (n, 0, 0)),
        ),
        compiler_params=pltpu.CompilerParams(
            dimension_semantics=("parallel",)),
        cost_estimate=pl.CostEstimate(
            flops=N * conv_flops,
            transcendentals=0,
            bytes_accessed=4 * N * Cin * HW + 2 * (3 * Cin * 3 * Cout
                                + N * Cout * HW) + 4 * (2 * HW * Cout
                                + N * Cout * 2)),
    )(x, w_cat, masks)

    y = pl.pallas_call(
        functools.partial(_norm_relu_kernel,
                          inv_count=1.0 / float(N * HW), eps=eps),
        out_shape=jax.ShapeDtypeStruct((N, HW, Cout), jnp.float32),
        grid=(N // bs2,),
        in_specs=[
            pl.BlockSpec((N, 2, Cout), lambda n: (0, 0, 0)),
            pl.BlockSpec((1, Cout), lambda n: (0, 0)),
            pl.BlockSpec((bs2, HW, Cout), lambda n: (n, 0, 0)),
        ],
        out_specs=pl.BlockSpec((bs2, HW, Cout), lambda n: (n, 0, 0)),
        compiler_params=pltpu.CompilerParams(
            dimension_semantics=("parallel",)),
        cost_estimate=pl.CostEstimate(
            flops=2 * N * Cout * HW,
            transcendentals=Cout,
            bytes_accessed=2 * N * Cout * HW + 4 * N * Cout * HW
                           + 4 * (N * Cout * 2 + Cout)),
    )(stats, beta.astype(jnp.float32).reshape(1, Cout), y_raw)

    return jnp.transpose(y.reshape(N, H, W, Cout), (0, 3, 1, 2))


def kernel(x_nchw, weight_oihw, beta):
    return _conv_block(x_nchw, weight_oihw, beta)


# speedup vs baseline: 1.1880x; 1.0235x over previous
import functools

import numpy as np

_IMGS_PER_STEP = 4

import jax
import jax.numpy as jnp
from jax import lax
from jax.experimental import pallas as pl
from jax.experimental.pallas import tpu as pltpu


def _edge_masks(H, W, C):
    col = np.arange(H * W) % W
    m = np.ones((2, H * W, 1), np.float32)
    m[0, col == 0, 0] = 0.0
    m[1, col == W - 1, 0] = 0.0
    return np.broadcast_to(m, (2, H * W, C)).copy()


def _conv_stats_kernel(x_ref, w_ref, m_ref, o_ref, s_ref, x3_ref, *, H, W):
    HW = H * W
    B = x_ref.shape[0]
    Cin = x_ref.shape[3]
    Cout = w_ref.shape[1] // 3
    zrow = jnp.zeros((W, Cin), jnp.float32)
    for b in range(B):
        x2 = x_ref[b].reshape(HW, Cin)
        x3_ref[b, :, 0:Cin] = jnp.concatenate(
            [zrow, x2[0:HW - W]], axis=0).astype(jnp.bfloat16)
        x3_ref[b, :, Cin:2 * Cin] = x2.astype(jnp.bfloat16)
        x3_ref[b, :, 2 * Cin:] = jnp.concatenate(
            [x2[W:HW], zrow], axis=0).astype(jnp.bfloat16)

        z = jnp.dot(x3_ref[b], w_ref[...],
                    preferred_element_type=jnp.float32)

        acc = z[:, Cout:2 * Cout]
        acc = acc + pltpu.roll(z[:, 0:Cout], 1, axis=0) * m_ref[0]
        acc = acc + pltpu.roll(z[:, 2 * Cout:], HW - 1, axis=0) * m_ref[1]

        o_ref[b] = acc.astype(jnp.bfloat16)
        s_ref[b, 0:1, :] = jnp.sum(acc, axis=0, keepdims=True)
        s_ref[b, 1:2, :] = jnp.sum(acc * acc, axis=0, keepdims=True)


def _norm_relu_kernel(s_ref, beta_ref, y_ref, o_ref, *, inv_count, eps):
    tot = jnp.sum(s_ref[...], axis=0)
    mean = tot[0:1, :] * inv_count
    var = jnp.maximum(tot[1:2, :] * inv_count - mean * mean, 0.0)
    scale = lax.rsqrt(var + eps)
    shift = beta_ref[...] - mean * scale
    for b in range(y_ref.shape[0]):
        y = y_ref[b].astype(jnp.float32)
        o_ref[b] = jnp.maximum(y * scale + shift, 0.0)


@jax.jit
def _conv_block(x_nchw, weight_oihw, beta):
    eps = 1e-5
    N, Cin, H, W = x_nchw.shape
    Cout = weight_oihw.shape[0]
    HW = H * W

    x = jnp.transpose(x_nchw, (0, 2, 3, 1))
    w_cat = jnp.transpose(weight_oihw.astype(jnp.float32),
                          (2, 1, 3, 0)).reshape(3 * Cin, 3 * Cout)
    w_cat = w_cat.astype(jnp.bfloat16)
    masks = jnp.asarray(_edge_masks(H, W, Cout), dtype=jnp.float32)

    conv_flops = 2 * Cout * 9 * Cin * HW
    bs = _IMGS_PER_STEP if N % _IMGS_PER_STEP == 0 else 1
    bs2 = 8 if N % 8 == 0 else 1

    y_raw, stats = pl.pallas_call(
        functools.partial(_conv_stats_kernel, H=H, W=W),
        out_shape=(
            jax.ShapeDtypeStruct((N, HW, Cout), jnp.bfloat16),
            jax.ShapeDtypeStruct((N, 2, Cout), jnp.float32),
        ),
        grid=(N // bs,),
        in_specs=[
            pl.BlockSpec((bs, H, W, Cin), lambda n: (n, 0, 0, 0)),
            pl.BlockSpec((3 * Cin, 3 * Cout), lambda n: (0, 0)),
            pl.BlockSpec((2, HW, Cout), lambda n: (0, 0, 0)),
        ],
        scratch_shapes=[pltpu.VMEM((bs, HW, 3 * Cin), jnp.bfloat16)],
        out_specs=(
            pl.BlockSpec((bs, HW, Cout), lambda n: (n, 0, 0)),
            pl.BlockSpec((bs, 2, Cout), lambda n: (n, 0, 0)),
        ),
        compiler_params=pltpu.CompilerParams(
            dimension_semantics=("parallel",)),
        cost_estimate=pl.CostEstimate(
            flops=N * conv_flops,
            transcendentals=0,
            bytes_accessed=4 * N * Cin * HW + 2 * (3 * Cin * 3 * Cout
                                + N * Cout * HW) + 4 * (2 * HW * Cout
                                + N * Cout * 2)),
    )(x, w_cat, masks)

    y = pl.pallas_call(
        functools.partial(_norm_relu_kernel,
                          inv_count=1.0 / float(N * HW), eps=eps),
        out_shape=jax.ShapeDtypeStruct((N, HW, Cout), jnp.float32),
        grid=(N // bs2,),
        in_specs=[
            pl.BlockSpec((N, 2, Cout), lambda n: (0, 0, 0)),
            pl.BlockSpec((1, Cout), lambda n: (0, 0)),
            pl.BlockSpec((bs2, HW, Cout), lambda n: (n, 0, 0)),
        ],
        out_specs=pl.BlockSpec((bs2, HW, Cout), lambda n: (n, 0, 0)),
        compiler_params=pltpu.CompilerParams(
            dimension_semantics=("parallel",)),
        cost_estimate=pl.CostEstimate(
            flops=2 * N * Cout * HW,
            transcendentals=Cout,
            bytes_accessed=2 * N * Cout * HW + 4 * N * Cout * HW
                           + 4 * (N * Cout * 2 + Cout)),
    )(stats, beta.astype(jnp.float32).reshape(1, Cout), y_raw)

    return jnp.transpose(y.reshape(N, H, W, Cout), (0, 3, 1, 2))


def kernel(x_nchw, weight_oihw, beta):
    return _conv_block(x_nchw, weight_oihw, beta)
